# Initial kernel scaffold; baseline (speedup 1.0000x reference)
import jax
import jax.numpy as jnp
from jax.experimental import pallas as pl


def kernel(x, W):
    raise NotImplementedError("write your pallas kernel here")


# speedup vs baseline: 4.9936x; 4.9936x over previous
import functools

import jax
import jax.numpy as jnp
from jax import lax
from jax.experimental import pallas as pl
from jax.experimental.pallas import tpu as pltpu
from jax.experimental.pallas import tpu_sc as plsc

D_MODEL = 64

_info = plsc.get_sparse_core_info()
_NC = _info.num_cores
_NS = _info.num_subcores
_NW = _NC * _NS


@functools.lru_cache(maxsize=None)
def _make_gather(B, C):
    assert B % _NW == 0
    b_per_w = B // _NW
    assert b_per_w % C == 0 and C % 8 == 0
    nchunks = b_per_w // C
    mesh = plsc.VectorSubcoreMesh(core_axis_name="c", subcore_axis_name="s")

    @functools.partial(
        pl.kernel,
        mesh=mesh,
        out_type=jax.ShapeDtypeStruct((B, D_MODEL), jnp.float32),
        scratch_types=[
            pltpu.VMEM((C,), jnp.int32),
            pltpu.VMEM((C, D_MODEL), jnp.float32),
            pltpu.SemaphoreType.DMA,
        ],
        compiler_params=pltpu.CompilerParams(use_tc_tiling_on_sc=False),
    )
    def gather_kernel(x_hbm, w_hbm, out_hbm, idx_v, rows_v, sem):
        wid = lax.axis_index("s") * _NC + lax.axis_index("c")
        base0 = wid * b_per_w

        def body(i, carry):
            base = base0 + i * C
            pltpu.sync_copy(x_hbm.at[pl.ds(base, C)], idx_v)
            pltpu.async_copy(w_hbm.at[idx_v], rows_v, sem).wait()
            pltpu.sync_copy(rows_v, out_hbm.at[pl.ds(base, C)])
            return carry

        lax.fori_loop(0, nchunks, body, 0)

    return gather_kernel


def kernel(x, W):
    S0, S1 = x.shape
    B = S0 * S1
    xf = x.reshape(B).astype(jnp.int32)
    out = _make_gather(B, 1024)(xf, W)
    return out.reshape(S0, S1, D_MODEL)


# speedup vs baseline: 5.0937x; 1.0200x over previous
import functools

import jax
import jax.numpy as jnp
from jax import lax
from jax.experimental import pallas as pl
from jax.experimental.pallas import tpu as pltpu
from jax.experimental.pallas import tpu_sc as plsc

D_MODEL = 64

_info = plsc.get_sparse_core_info()
_NC = _info.num_cores
_NS = _info.num_subcores
_NW = _NC * _NS


@functools.lru_cache(maxsize=None)
def _make_gather(B, C):
    assert B % _NW == 0
    b_per_w = B // _NW
    assert b_per_w % C == 0 and C % 8 == 0
    nchunks = b_per_w // C
    assert nchunks >= 2 and nchunks % 2 == 0
    mesh = plsc.VectorSubcoreMesh(core_axis_name="c", subcore_axis_name="s")

    @functools.partial(
        pl.kernel,
        mesh=mesh,
        out_type=jax.ShapeDtypeStruct((B, D_MODEL), jnp.float32),
        scratch_types=[
            pltpu.VMEM((C,), jnp.int32),
            pltpu.VMEM((C,), jnp.int32),
            pltpu.VMEM((C, D_MODEL), jnp.float32),
            pltpu.VMEM((C, D_MODEL), jnp.float32),
            pltpu.SemaphoreType.DMA,
            pltpu.SemaphoreType.DMA,
            pltpu.SemaphoreType.DMA,
            pltpu.SemaphoreType.DMA,
        ],
        compiler_params=pltpu.CompilerParams(use_tc_tiling_on_sc=False),
    )
    def gather_kernel(x_hbm, w_hbm, out_hbm, idx0, idx1, rows0, rows1,
                      g0, g1, o0, o1):
        idx = (idx0, idx1)
        rows = (rows0, rows1)
        gs = (g0, g1)
        os_ = (o0, o1)
        wid = lax.axis_index("s") * _NC + lax.axis_index("c")
        base0 = wid * b_per_w

        def load_and_gather(i, b):
            base = base0 + i * C
            pltpu.sync_copy(x_hbm.at[pl.ds(base, C)], idx[b])
            pltpu.async_copy(w_hbm.at[idx[b]], rows[b], gs[b])

        def gather_wait(b):
            pltpu.make_async_copy(w_hbm.at[idx[b]], rows[b], gs[b]).wait()

        def out_start(i, b):
            base = base0 + i * C
            pltpu.async_copy(rows[b], out_hbm.at[pl.ds(base, C)], os_[b])

        def out_wait(b):
            pltpu.make_async_copy(
                rows[b], out_hbm.at[pl.ds(base0, C)], os_[b]).wait()

        load_and_gather(0, 0)
        load_and_gather(1, 1)
        gather_wait(0)
        out_start(0, 0)

        def body(j, carry):
            i1 = 2 * j + 1
            out_wait(0)
            load_and_gather(i1 + 1, 0)
            gather_wait(1)
            out_start(i1, 1)
            i2 = i1 + 1
            out_wait(1)
            load_and_gather(i2 + 1, 1)
            gather_wait(0)
            out_start(i2, 0)
            return carry

        lax.fori_loop(0, (nchunks - 2) // 2, body, 0)

        gather_wait(1)
        out_start(nchunks - 1, 1)
        out_wait(0)
        out_wait(1)

    return gather_kernel


def kernel(x, W):
    S0, S1 = x.shape
    B = S0 * S1
    xf = x.reshape(B).astype(jnp.int32)
    out = _make_gather(B, 800)(xf, W)
    return out.reshape(S0, S1, D_MODEL)
